# baseline (device time: 30010 ns/iter reference)
import jax
import jax.numpy as jnp
from jax import lax
from jax.experimental import pallas as pl
from jax.experimental.pallas import tpu as pltpu

N_DEV = 8
B, Sq, D = 2, 256, 768
Hq, Hkv, Dh = 8, 2, 64
G = Hq // Hkv
SCALE = 0.125
ROWS = B * G * Sq
PARTS = ((0, 688), (688, 1376), (1376, ROWS))
SCHED = ((1, 3, 4), (3, 4, 1), (4, 1, 3))
L_SCHED = (4, 1, 3)


def kernel(x, Wq, Wo, K_ext, V_ext):
    Skv = K_ext.shape[1]
    x2 = x.reshape(B * Sq, D)
    kt = K_ext.transpose(0, 2, 3, 1)
    vt = V_ext.transpose(0, 2, 3, 1)

    def body(x_ref, wq_ref, wo_hbm, kt_ref, vt_ref, out_hbm,
             send_buf, recv_buf, l_send, l_recv, wo_vmem, out_vmem,
             send_sems, recv_sems, l_send_sems, l_recv_sems,
             wo_dma_sem, out_dma_sems):
        my = lax.axis_index("i")

        wo_dma = pltpu.make_async_copy(wo_hbm, wo_vmem, wo_dma_sem)
        wo_dma.start()

        q = jnp.dot(x_ref[...], wq_ref[...],
                    preferred_element_type=jnp.float32)
        ones_row = jnp.ones((1, Skv), jnp.float32)

        def partial(b, g):
            qg = jnp.concatenate(
                [q[b * Sq:(b + 1) * Sq, (g * G + j) * Dh:(g * G + j + 1) * Dh]
                 for j in range(G)], axis=0)
            s = lax.dot_general(
                qg, kt_ref[b, g], (((1,), (0,)), ((), ())),
                preferred_element_type=jnp.float32) * SCALE
            p = jnp.exp(s)
            acc = lax.dot_general(
                p, vt_ref[b, g], (((1,), (1,)), ((), ())),
                preferred_element_type=jnp.float32)
            send_buf[b * G * Sq:(b + 1) * G * Sq, g * Dh:(g + 1) * Dh] = (
                acc.astype(jnp.bfloat16))
            bg = b * Hkv + g
            l_send[bg:bg + 1, :] = lax.dot_general(
                ones_row, p, (((1,), (1,)), ((), ())),
                preferred_element_type=jnp.float32)

        def mk_rdma(ph, part):
            r0, r1 = PARTS[part]
            return pltpu.make_async_remote_copy(
                src_ref=send_buf.at[pl.ds(r0, r1 - r0), :],
                dst_ref=recv_buf.at[ph, pl.ds(r0, r1 - r0), :],
                send_sem=send_sems.at[ph, part],
                recv_sem=recv_sems.at[ph, part],
                device_id=(my ^ SCHED[part][ph],),
                device_id_type=pl.DeviceIdType.MESH)

        def mk_l_rdma(ph):
            return pltpu.make_async_remote_copy(
                src_ref=l_send, dst_ref=l_recv.at[ph],
                send_sem=l_send_sems.at[ph], recv_sem=l_recv_sems.at[ph],
                device_id=(my ^ L_SCHED[ph],),
                device_id_type=pl.DeviceIdType.MESH)

        partial(0, 0)
        partial(0, 1)
        barrier = pltpu.get_barrier_semaphore()
        for mask in (1, 3, 4):
            pl.semaphore_signal(barrier, inc=1, device_id=(my ^ mask,),
                                device_id_type=pl.DeviceIdType.MESH)
        pl.semaphore_wait(barrier, 3)

        rdma0 = mk_rdma(0, 0)
        rdma0.start()
        partial(1, 0)
        partial(1, 1)
        chains = [rdma0, mk_rdma(0, 1), mk_rdma(0, 2), mk_l_rdma(0)]
        for r in chains[1:]:
            r.start()

        def merge(ph, part):
            r0, r1 = PARTS[part]
            send_buf[r0:r1, :] = send_buf[r0:r1, :] + recv_buf[ph, r0:r1, :]

        def l_merge(ph):
            l_send[...] = l_send[...] + l_recv[ph]

        for ph in range(2):
            nxt = []
            for part in range(3):
                chains[part].wait()
                merge(ph, part)
                r = mk_rdma(ph + 1, part)
                r.start()
                nxt.append(r)
            chains[3].wait()
            l_merge(ph)
            r = mk_l_rdma(ph + 1)
            r.start()
            nxt.append(r)
            chains = nxt

        n = G * Sq
        ri = lax.broadcasted_iota(jnp.int32, (n, n), 0)
        ci = lax.broadcasted_iota(jnp.int32, (n, n), 1)
        eye = jnp.where(ri == ci, 1.0, 0.0).astype(jnp.float32)

        for part in range(3):
            chains[part].wait()
            merge(2, part)
        chains[3].wait()
        l_merge(2)

        red = send_buf[...].astype(jnp.float32)
        lcol = lax.dot_general(
            eye, l_send[...], (((1,), (1,)), ((), ())),
            preferred_element_type=jnp.float32)
        wo_dma.wait()
        out_dmas = []
        for b in range(B):
            cols = []
            for hq in range(Hq):
                g, j = hq // G, hq % G
                r0 = b * n + j * Sq
                bg = b * Hkv + g
                o = (red[r0:r0 + Sq, g * Dh:(g + 1) * Dh]
                     / lcol[j * Sq:(j + 1) * Sq, bg:bg + 1])
                cols.append(o)
            row = jnp.concatenate(cols, axis=1)
            out_vmem[b * Sq:(b + 1) * Sq, :] = jnp.dot(
                row, wo_vmem[...], preferred_element_type=jnp.float32)
            od = pltpu.make_async_copy(
                out_vmem.at[pl.ds(b * Sq, Sq), :],
                out_hbm.at[pl.ds(b * Sq, Sq), :],
                out_dma_sems.at[b])
            od.start()
            out_dmas.append(od)
        for od in out_dmas:
            od.wait()

    out = pl.pallas_call(
        body,
        out_shape=jax.ShapeDtypeStruct((B * Sq, D), jnp.float32),
        in_specs=[
            pl.BlockSpec(memory_space=pltpu.VMEM),
            pl.BlockSpec(memory_space=pltpu.VMEM),
            pl.BlockSpec(memory_space=pl.ANY),
            pl.BlockSpec(memory_space=pltpu.VMEM),
            pl.BlockSpec(memory_space=pltpu.VMEM),
        ],
        out_specs=pl.BlockSpec(memory_space=pl.ANY),
        scratch_shapes=[
            pltpu.VMEM((ROWS, 128), jnp.bfloat16),
            pltpu.VMEM((3, ROWS, 128), jnp.bfloat16),
            pltpu.VMEM((B * Hkv, G * Sq), jnp.float32),
            pltpu.VMEM((3, B * Hkv, G * Sq), jnp.float32),
            pltpu.VMEM((Hq * Dh, D), jnp.float32),
            pltpu.VMEM((B * Sq, D), jnp.float32),
            pltpu.SemaphoreType.DMA((3, 3)),
            pltpu.SemaphoreType.DMA((3, 3)),
            pltpu.SemaphoreType.DMA((3,)),
            pltpu.SemaphoreType.DMA((3,)),
            pltpu.SemaphoreType.DMA(()),
            pltpu.SemaphoreType.DMA((B,)),
        ],
        compiler_params=pltpu.CompilerParams(collective_id=0),
    )(x2, Wq, Wo, kt, vt)
    return out.reshape(B, Sq, D)


# device time: 29728 ns/iter; 1.0095x vs baseline; 1.0095x over previous
import jax
import jax.numpy as jnp
from jax import lax
from jax.experimental import pallas as pl
from jax.experimental.pallas import tpu as pltpu

N_DEV = 8
B, Sq, D = 2, 256, 768
Hq, Hkv, Dh = 8, 2, 64
G = Hq // Hkv
SCALE = 0.125
ROWS = B * G * Sq
PARTS = ((0, 688), (688, 1376), (1376, ROWS))
SCHED = ((1, 3, 4), (3, 4, 1), (4, 1, 3))
L_SCHED = (4, 1, 3)


def kernel(x, Wq, Wo, K_ext, V_ext):
    Skv = K_ext.shape[1]
    x2 = x.reshape(B * Sq, D)
    kt = K_ext.transpose(0, 2, 3, 1)
    vt = V_ext.transpose(0, 2, 3, 1)

    def body(x_ref, wq_ref, wo_hbm, kt_hbm, vt_hbm, out_ref,
             send_buf, recv_buf, l_send, l_recv, kt_vmem, vt_vmem, wo_vmem,
             send_sems, recv_sems, l_send_sems, l_recv_sems,
             kv_dma_sems, wo_dma_sem):
        my = lax.axis_index("i")

        k_dma = pltpu.make_async_copy(kt_hbm, kt_vmem, kv_dma_sems.at[0])
        v_dma = pltpu.make_async_copy(vt_hbm, vt_vmem, kv_dma_sems.at[1])
        wo_dma = pltpu.make_async_copy(wo_hbm, wo_vmem, wo_dma_sem)
        k_dma.start()
        v_dma.start()
        wo_dma.start()

        q = jnp.dot(x_ref[...], wq_ref[...],
                    preferred_element_type=jnp.float32)
        ones_row = jnp.ones((1, Skv), jnp.float32)
        k_dma.wait()
        v_dma.wait()

        def partial(b, g):
            qg = jnp.concatenate(
                [q[b * Sq:(b + 1) * Sq, (g * G + j) * Dh:(g * G + j + 1) * Dh]
                 for j in range(G)], axis=0)
            s = lax.dot_general(
                qg, kt_vmem[b, g], (((1,), (0,)), ((), ())),
                preferred_element_type=jnp.float32) * SCALE
            p = jnp.exp(s)
            acc = lax.dot_general(
                p, vt_vmem[b, g], (((1,), (1,)), ((), ())),
                preferred_element_type=jnp.float32)
            send_buf[b * G * Sq:(b + 1) * G * Sq, g * Dh:(g + 1) * Dh] = (
                acc.astype(jnp.bfloat16))
            bg = b * Hkv + g
            l_send[bg:bg + 1, :] = lax.dot_general(
                ones_row, p, (((1,), (1,)), ((), ())),
                preferred_element_type=jnp.float32)

        def mk_rdma(ph, part):
            r0, r1 = PARTS[part]
            return pltpu.make_async_remote_copy(
                src_ref=send_buf.at[pl.ds(r0, r1 - r0), :],
                dst_ref=recv_buf.at[ph, pl.ds(r0, r1 - r0), :],
                send_sem=send_sems.at[ph, part],
                recv_sem=recv_sems.at[ph, part],
                device_id=(my ^ SCHED[part][ph],),
                device_id_type=pl.DeviceIdType.MESH)

        def mk_l_rdma(ph):
            return pltpu.make_async_remote_copy(
                src_ref=l_send, dst_ref=l_recv.at[ph],
                send_sem=l_send_sems.at[ph], recv_sem=l_recv_sems.at[ph],
                device_id=(my ^ L_SCHED[ph],),
                device_id_type=pl.DeviceIdType.MESH)

        partial(0, 0)
        partial(0, 1)
        barrier = pltpu.get_barrier_semaphore()
        for mask in (1, 3, 4):
            pl.semaphore_signal(barrier, inc=1, device_id=(my ^ mask,),
                                device_id_type=pl.DeviceIdType.MESH)
        pl.semaphore_wait(barrier, 3)

        rdma0 = mk_rdma(0, 0)
        rdma0.start()
        partial(1, 0)
        partial(1, 1)
        chains = [rdma0, mk_rdma(0, 1), mk_rdma(0, 2), mk_l_rdma(0)]
        for r in chains[1:]:
            r.start()

        def merge(ph, part):
            r0, r1 = PARTS[part]
            send_buf[r0:r1, :] = send_buf[r0:r1, :] + recv_buf[ph, r0:r1, :]

        def l_merge(ph):
            l_send[...] = l_send[...] + l_recv[ph]

        for ph in range(2):
            nxt = []
            for part in range(3):
                chains[part].wait()
                merge(ph, part)
                r = mk_rdma(ph + 1, part)
                r.start()
                nxt.append(r)
            chains[3].wait()
            l_merge(ph)
            r = mk_l_rdma(ph + 1)
            r.start()
            nxt.append(r)
            chains = nxt

        n = G * Sq
        ri = lax.broadcasted_iota(jnp.int32, (n, n), 0)
        ci = lax.broadcasted_iota(jnp.int32, (n, n), 1)
        eye = jnp.where(ri == ci, 1.0, 0.0).astype(jnp.float32)

        for part in range(3):
            chains[part].wait()
            merge(2, part)
        chains[3].wait()
        l_merge(2)

        red = send_buf[...].astype(jnp.float32)
        lcol = lax.dot_general(
            eye, l_send[...], (((1,), (1,)), ((), ())),
            preferred_element_type=jnp.float32)
        wo_dma.wait()
        for b in range(B):
            cols = []
            for hq in range(Hq):
                g, j = hq // G, hq % G
                r0 = b * n + j * Sq
                bg = b * Hkv + g
                o = (red[r0:r0 + Sq, g * Dh:(g + 1) * Dh]
                     / lcol[j * Sq:(j + 1) * Sq, bg:bg + 1])
                cols.append(o)
            row = jnp.concatenate(cols, axis=1)
            out_ref[b] = jnp.dot(
                row, wo_vmem[...], preferred_element_type=jnp.float32)

    return pl.pallas_call(
        body,
        out_shape=jax.ShapeDtypeStruct((B, Sq, D), jnp.float32),
        in_specs=[
            pl.BlockSpec(memory_space=pltpu.VMEM),
            pl.BlockSpec(memory_space=pltpu.VMEM),
            pl.BlockSpec(memory_space=pl.ANY),
            pl.BlockSpec(memory_space=pl.ANY),
            pl.BlockSpec(memory_space=pl.ANY),
        ],
        out_specs=pl.BlockSpec(memory_space=pltpu.VMEM),
        scratch_shapes=[
            pltpu.VMEM((ROWS, 128), jnp.bfloat16),
            pltpu.VMEM((3, ROWS, 128), jnp.bfloat16),
            pltpu.VMEM((B * Hkv, G * Sq), jnp.float32),
            pltpu.VMEM((3, B * Hkv, G * Sq), jnp.float32),
            pltpu.VMEM((B, Hkv, Dh, 512), jnp.float32),
            pltpu.VMEM((B, Hkv, Dh, 512), jnp.float32),
            pltpu.VMEM((Hq * Dh, D), jnp.float32),
            pltpu.SemaphoreType.DMA((3, 3)),
            pltpu.SemaphoreType.DMA((3, 3)),
            pltpu.SemaphoreType.DMA((3,)),
            pltpu.SemaphoreType.DMA((3,)),
            pltpu.SemaphoreType.DMA((2,)),
            pltpu.SemaphoreType.DMA(()),
        ],
        compiler_params=pltpu.CompilerParams(collective_id=0),
    )(x2, Wq, Wo, kt, vt)


# device time: 28226 ns/iter; 1.0632x vs baseline; 1.0532x over previous
import jax
import jax.numpy as jnp
from jax import lax
from jax.experimental import pallas as pl
from jax.experimental.pallas import tpu as pltpu

N_DEV = 8
B, Sq, D = 2, 256, 768
Hq, Hkv, Dh = 8, 2, 64
G = Hq // Hkv
SCALE = 0.125
ROWS = B * G * Sq
PARTS = ((0, 688), (688, 1376), (1376, ROWS))
SCHED = ((1, 3, 4), (3, 4, 1), (4, 1, 3))
L_SCHED = (4, 1, 3)


def kernel(x, Wq, Wo, K_ext, V_ext):
    Skv = K_ext.shape[1]
    x2 = x.reshape(B * Sq, D)
    k2 = K_ext.reshape(B, Skv, Hkv * Dh)
    v2 = V_ext.reshape(B, Skv, Hkv * Dh)

    def body(x_ref, wq_ref, wo_ref, k_ref, v_ref, out_ref,
             send_buf, recv_buf, l_send, l_recv,
             send_sems, recv_sems, l_send_sems, l_recv_sems):
        my = lax.axis_index("i")

        q = jnp.dot(x_ref[...], wq_ref[...],
                    preferred_element_type=jnp.float32)
        ones_row = jnp.ones((1, Skv), jnp.float32)

        def partial(b, g):
            qg = jnp.concatenate(
                [q[b * Sq:(b + 1) * Sq, (g * G + j) * Dh:(g * G + j + 1) * Dh]
                 for j in range(G)], axis=0)
            kg = k_ref[b, :, g * Dh:(g + 1) * Dh]
            vg = v_ref[b, :, g * Dh:(g + 1) * Dh]
            s = lax.dot_general(
                qg, kg, (((1,), (1,)), ((), ())),
                preferred_element_type=jnp.float32) * SCALE
            p = jnp.exp(s)
            acc = jnp.dot(p, vg, preferred_element_type=jnp.float32)
            send_buf[b * G * Sq:(b + 1) * G * Sq, g * Dh:(g + 1) * Dh] = (
                acc.astype(jnp.bfloat16))
            bg = b * Hkv + g
            l_send[bg:bg + 1, :] = lax.dot_general(
                ones_row, p, (((1,), (1,)), ((), ())),
                preferred_element_type=jnp.float32)

        def mk_rdma(ph, part):
            r0, r1 = PARTS[part]
            return pltpu.make_async_remote_copy(
                src_ref=send_buf.at[pl.ds(r0, r1 - r0), :],
                dst_ref=recv_buf.at[ph, pl.ds(r0, r1 - r0), :],
                send_sem=send_sems.at[ph, part],
                recv_sem=recv_sems.at[ph, part],
                device_id=(my ^ SCHED[part][ph],),
                device_id_type=pl.DeviceIdType.MESH)

        def mk_l_rdma(ph):
            return pltpu.make_async_remote_copy(
                src_ref=l_send, dst_ref=l_recv.at[ph],
                send_sem=l_send_sems.at[ph], recv_sem=l_recv_sems.at[ph],
                device_id=(my ^ L_SCHED[ph],),
                device_id_type=pl.DeviceIdType.MESH)

        partial(0, 0)
        partial(0, 1)
        barrier = pltpu.get_barrier_semaphore()
        for mask in (1, 3, 4):
            pl.semaphore_signal(barrier, inc=1, device_id=(my ^ mask,),
                                device_id_type=pl.DeviceIdType.MESH)
        pl.semaphore_wait(barrier, 3)

        rdma0 = mk_rdma(0, 0)
        rdma0.start()
        partial(1, 0)
        partial(1, 1)
        chains = [rdma0, mk_rdma(0, 1), mk_rdma(0, 2), mk_l_rdma(0)]
        for r in chains[1:]:
            r.start()

        def merge(ph, part):
            r0, r1 = PARTS[part]
            send_buf[r0:r1, :] = send_buf[r0:r1, :] + recv_buf[ph, r0:r1, :]

        def l_merge(ph):
            l_send[...] = l_send[...] + l_recv[ph]

        for ph in range(2):
            nxt = []
            for part in range(3):
                chains[part].wait()
                merge(ph, part)
                r = mk_rdma(ph + 1, part)
                r.start()
                nxt.append(r)
            chains[3].wait()
            l_merge(ph)
            r = mk_l_rdma(ph + 1)
            r.start()
            nxt.append(r)
            chains = nxt

        n = G * Sq
        ri = lax.broadcasted_iota(jnp.int32, (n, n), 0)
        ci = lax.broadcasted_iota(jnp.int32, (n, n), 1)
        eye = jnp.where(ri == ci, 1.0, 0.0).astype(jnp.float32)

        for part in range(3):
            chains[part].wait()
            merge(2, part)
        chains[3].wait()
        l_merge(2)

        red = send_buf[...].astype(jnp.float32)
        lcol = lax.dot_general(
            eye, l_send[...], (((1,), (1,)), ((), ())),
            preferred_element_type=jnp.float32)
        for b in range(B):
            cols = []
            for hq in range(Hq):
                g, j = hq // G, hq % G
                r0 = b * n + j * Sq
                bg = b * Hkv + g
                o = (red[r0:r0 + Sq, g * Dh:(g + 1) * Dh]
                     / lcol[j * Sq:(j + 1) * Sq, bg:bg + 1])
                cols.append(o)
            row = jnp.concatenate(cols, axis=1)
            out_ref[b * Sq:(b + 1) * Sq, :] = jnp.dot(
                row, wo_ref[...], preferred_element_type=jnp.float32)

    out = pl.pallas_call(
        body,
        out_shape=jax.ShapeDtypeStruct((B * Sq, D), jnp.float32),
        in_specs=[pl.BlockSpec(memory_space=pltpu.VMEM)] * 5,
        out_specs=pl.BlockSpec(memory_space=pltpu.VMEM),
        scratch_shapes=[
            pltpu.VMEM((ROWS, 128), jnp.bfloat16),
            pltpu.VMEM((3, ROWS, 128), jnp.bfloat16),
            pltpu.VMEM((B * Hkv, G * Sq), jnp.float32),
            pltpu.VMEM((3, B * Hkv, G * Sq), jnp.float32),
            pltpu.SemaphoreType.DMA((3, 3)),
            pltpu.SemaphoreType.DMA((3, 3)),
            pltpu.SemaphoreType.DMA((3,)),
            pltpu.SemaphoreType.DMA((3,)),
        ],
        compiler_params=pltpu.CompilerParams(collective_id=0),
    )(x2, Wq, Wo, k2, v2)
    return out.reshape(B, Sq, D)
